# Initial kernel scaffold; baseline (speedup 1.0000x reference)
import jax
import jax.numpy as jnp
from jax.experimental import pallas as pl


def kernel(vertices):
    raise NotImplementedError("write your pallas kernel here")


# speedup vs baseline: 3.4352x; 3.4352x over previous
import functools

import jax
import jax.numpy as jnp
from jax import lax
from jax.experimental import pallas as pl
from jax.experimental.pallas import tpu as pltpu
from jax.experimental.pallas import tpu_sc as plsc

K = 16
C = 3
B_, N_ = 8, 4096
R = 256
CHUNK = 128
NCH = N_ // CHUNK
MAXI = 0x7FFFFFFF


def _topk_body(vb_ref, vt_ref, idx_ref):
    vb = vb_ref[0]
    vt = vt_ref[0]
    g = jnp.dot(vb, vt, preferred_element_type=jnp.float32)
    sqb = jnp.sum(vb * vb, axis=1, keepdims=True)
    sqf = jnp.sum(vt * vt, axis=0, keepdims=True)
    dist = sqb - 2.0 * g + sqf
    bits = lax.bitcast_convert_type(jnp.maximum(dist, 0.0), jnp.int32)
    b3 = bits.reshape(R, NCH, CHUNK)
    li3 = lax.broadcasted_iota(jnp.int32, (R, NCH, CHUNK), 2)
    p = (b3 & jnp.int32(-CHUNK)) | li3
    fi = lax.broadcasted_iota(jnp.int32, (R, NCH, CHUNK), 1) * CHUNK + li3
    c_iota = lax.broadcasted_iota(jnp.int32, (R, NCH), 1)
    a_iota = lax.broadcasted_iota(jnp.int32, (R, 32), 1)

    def it(t, carry):
        p, acc = carry
        m2 = jnp.min(p, axis=2)
        m = jnp.min(m2, axis=1, keepdims=True)
        cstar = jnp.min(jnp.where(m2 == m, c_iota, jnp.int32(MAXI)),
                        axis=1, keepdims=True)
        gidx = cstar * CHUNK + (m & (CHUNK - 1))
        p = jnp.where(fi == gidx[:, :, None], jnp.int32(MAXI), p)
        acc = jnp.where(a_iota == t, gidx, acc)
        return (p, acc)

    _, acc = lax.fori_loop(0, K + 1, it,
                           (p, jnp.zeros((R, 32), jnp.int32)))
    idx_ref[0] = acc[:, 1:K + 1]


def _tc_topk(vertices, verts_t):
    b, n, _ = vertices.shape
    return pl.pallas_call(
        _topk_body,
        grid=(b, n // R),
        in_specs=[
            pl.BlockSpec((1, R, C), lambda i, j: (i, j, 0)),
            pl.BlockSpec((1, C, n), lambda i, j: (i, 0, 0)),
        ],
        out_specs=pl.BlockSpec((1, R, K), lambda i, j: (i, j, 0)),
        out_shape=jax.ShapeDtypeStruct((b, n, K), jnp.int32),
    )(vertices, verts_t)


ROWS_PER_W = N_ * B_ // 32
SUB = 256
NSUB = ROWS_PER_W // SUB


def _sc_gather(vt_flat, idx_flat):
    mesh = plsc.VectorSubcoreMesh(core_axis_name="c", subcore_axis_name="s")

    @functools.partial(
        pl.kernel,
        mesh=mesh,
        out_type=jax.ShapeDtypeStruct((B_ * N_ * K * 2 * C,), jnp.float32),
        compiler_params=pltpu.CompilerParams(needs_layout_passes=False),
        scratch_types=[
            pltpu.VMEM((N_,), jnp.float32),
            pltpu.VMEM((N_,), jnp.float32),
            pltpu.VMEM((N_,), jnp.float32),
            pltpu.VMEM((SUB * K,), jnp.int32),
            pltpu.VMEM((SUB * K * 2 * C,), jnp.float32),
        ],
    )
    def body(vt_hbm, idx_hbm, out_hbm, vx, vy, vz, idxb, outb):
        wid = lax.axis_index("c") * 16 + lax.axis_index("s")
        b = wid // 4
        q = wid % 4
        pltpu.sync_copy(vt_hbm.at[pl.ds((b * 3 + 0) * N_, N_)], vx)
        pltpu.sync_copy(vt_hbm.at[pl.ds((b * 3 + 1) * N_, N_)], vy)
        pltpu.sync_copy(vt_hbm.at[pl.ds((b * 3 + 2) * N_, N_)], vz)
        i6 = lax.iota(jnp.int32, 16) * jnp.int32(2 * C)

        for s in range(NSUB):
            row0 = q * ROWS_PER_W + s * SUB
            goff = (b * N_ + row0) * K
            pltpu.sync_copy(idx_hbm.at[pl.ds(goff, SUB * K)], idxb)

            def rb(r, carry):
                iv = idxb[pl.ds(r * K, K)]
                civ = jnp.broadcast_to(row0 + r, (K,)).astype(jnp.int32)
                nx = plsc.load_gather(vx, [iv])
                ny = plsc.load_gather(vy, [iv])
                nz = plsc.load_gather(vz, [iv])
                cx = plsc.load_gather(vx, [civ])
                cy = plsc.load_gather(vy, [civ])
                cz = plsc.load_gather(vz, [civ])
                off = r * jnp.int32(K * 2 * C) + i6
                plsc.store_scatter(outb, [off + 0], nx - cx)
                plsc.store_scatter(outb, [off + 1], ny - cy)
                plsc.store_scatter(outb, [off + 2], nz - cz)
                plsc.store_scatter(outb, [off + 3], cx)
                plsc.store_scatter(outb, [off + 4], cy)
                plsc.store_scatter(outb, [off + 5], cz)
                return carry

            lax.fori_loop(0, SUB, rb, 0)
            pltpu.sync_copy(outb, out_hbm.at[pl.ds(goff * 2 * C, SUB * K * 2 * C)])

    return body(vt_flat, idx_flat)


def kernel(vertices):
    b, n, c = vertices.shape
    verts_t = jnp.transpose(vertices, (0, 2, 1))
    idx = _tc_topk(vertices, verts_t)
    out_flat = _sc_gather(verts_t.reshape(-1), idx.reshape(-1))
    return out_flat.reshape(b, n, K, 2 * c)


# speedup vs baseline: 10.3977x; 3.0268x over previous
import functools

import jax
import jax.numpy as jnp
from jax import lax
from jax.experimental import pallas as pl
from jax.experimental.pallas import tpu as pltpu
from jax.experimental.pallas import tpu_sc as plsc

K = 16
C = 3
B_, N_ = 8, 4096
RB = 128
CHUNK = 128
NCH = N_ // CHUNK
MAXI = 0x7FFFFFFF


def _topk_body(vall_ref, vtb_ref, idx_ref):
    va = vall_ref[0]
    vtb = vtb_ref[0]
    g = jnp.dot(va, vtb, preferred_element_type=jnp.float32)
    sqa = jnp.sum(va * va, axis=1, keepdims=True)
    sqb = jnp.sum(vtb * vtb, axis=0, keepdims=True)
    dist = sqa - 2.0 * g + sqb
    bits = lax.bitcast_convert_type(jnp.maximum(dist, 0.0), jnp.int32)
    b3 = bits.reshape(NCH, CHUNK, RB)
    li3 = lax.broadcasted_iota(jnp.int32, (NCH, CHUNK, RB), 1)
    p = (b3 & jnp.int32(-CHUNK)) | li3
    fi = lax.broadcasted_iota(jnp.int32, (NCH, CHUNK, RB), 0) * CHUNK + li3
    ci = lax.broadcasted_iota(jnp.int32, (NCH, RB), 0)
    ti = lax.broadcasted_iota(jnp.int32, (K, RB), 0)

    def it(t, carry):
        p, acc = carry
        m2 = jnp.min(p, axis=1)
        m = jnp.min(m2, axis=0, keepdims=True)
        cstar = jnp.min(jnp.where(m2 == m, ci, jnp.int32(MAXI)),
                        axis=0, keepdims=True)
        gidx = cstar * CHUNK + (m & (CHUNK - 1))
        p = jnp.where(fi == gidx.reshape(1, 1, RB), jnp.int32(MAXI), p)
        acc = jnp.where(ti == t - 1, gidx, acc)
        return (p, acc)

    _, acc = lax.fori_loop(0, K + 1, it,
                           (p, jnp.zeros((K, RB), jnp.int32)))
    idx_ref[0] = acc


def _tc_topk(vertices, verts_t):
    b, n, _ = vertices.shape
    return pl.pallas_call(
        _topk_body,
        grid=(b, n // RB),
        in_specs=[
            pl.BlockSpec((1, n, C), lambda i, j: (i, 0, 0)),
            pl.BlockSpec((1, C, RB), lambda i, j: (i, 0, j)),
        ],
        out_specs=pl.BlockSpec((1, K, RB), lambda i, j: (i, 0, j)),
        out_shape=jax.ShapeDtypeStruct((b, K, n), jnp.int32),
    )(vertices, verts_t)


ROWS_PER_W = N_ * B_ // 32
SUB = 256
NSUB = ROWS_PER_W // SUB


def _sc_gather(vt_flat, idx_t):
    mesh = plsc.VectorSubcoreMesh(core_axis_name="c", subcore_axis_name="s")

    @functools.partial(
        pl.kernel,
        mesh=mesh,
        out_type=jax.ShapeDtypeStruct((B_ * N_ * K * 2 * C,), jnp.float32),
        compiler_params=pltpu.CompilerParams(needs_layout_passes=False),
        scratch_types=[
            pltpu.VMEM((N_,), jnp.float32),
            pltpu.VMEM((N_,), jnp.float32),
            pltpu.VMEM((N_,), jnp.float32),
            pltpu.VMEM((K, SUB), jnp.int32),
            pltpu.VMEM((SUB * K * 2 * C,), jnp.float32),
        ],
    )
    def body(vt_hbm, idx_hbm, out_hbm, vx, vy, vz, idxb, outb):
        wid = lax.axis_index("c") * 16 + lax.axis_index("s")
        b = wid // 4
        q = wid % 4
        pltpu.sync_copy(vt_hbm.at[pl.ds((b * 3 + 0) * N_, N_)], vx)
        pltpu.sync_copy(vt_hbm.at[pl.ds((b * 3 + 1) * N_, N_)], vy)
        pltpu.sync_copy(vt_hbm.at[pl.ds((b * 3 + 2) * N_, N_)], vz)
        i6 = lax.iota(jnp.int32, 16) * jnp.int32(2 * C)
        t_iota = lax.iota(jnp.int32, 16)

        for s in range(NSUB):
            row0 = q * ROWS_PER_W + s * SUB
            pltpu.sync_copy(idx_hbm.at[b, pl.ds(0, K), pl.ds(row0, SUB)],
                            idxb)

            def rb(r, carry):
                rv = jnp.broadcast_to(r, (16,)).astype(jnp.int32)
                iv = plsc.load_gather(idxb, [t_iota, rv])
                civ = jnp.broadcast_to(row0 + r, (16,)).astype(jnp.int32)
                nx = plsc.load_gather(vx, [iv])
                ny = plsc.load_gather(vy, [iv])
                nz = plsc.load_gather(vz, [iv])
                cx = plsc.load_gather(vx, [civ])
                cy = plsc.load_gather(vy, [civ])
                cz = plsc.load_gather(vz, [civ])
                off = r * jnp.int32(K * 2 * C) + i6
                plsc.store_scatter(outb, [off + 0], nx - cx)
                plsc.store_scatter(outb, [off + 1], ny - cy)
                plsc.store_scatter(outb, [off + 2], nz - cz)
                plsc.store_scatter(outb, [off + 3], cx)
                plsc.store_scatter(outb, [off + 4], cy)
                plsc.store_scatter(outb, [off + 5], cz)
                return carry

            lax.fori_loop(0, SUB, rb, 0)
            goff = (b * N_ + row0) * K * 2 * C
            pltpu.sync_copy(outb, out_hbm.at[pl.ds(goff, SUB * K * 2 * C)])

    return body(vt_flat, idx_t)


def kernel(vertices):
    b, n, c = vertices.shape
    verts_t = jnp.transpose(vertices, (0, 2, 1))
    idx_t = _tc_topk(vertices, verts_t)
    out_flat = _sc_gather(verts_t.reshape(-1), idx_t)
    return out_flat.reshape(b, n, K, 2 * c)


# speedup vs baseline: 10.5112x; 1.0109x over previous
import functools

import jax
import jax.numpy as jnp
from jax import lax
from jax.experimental import pallas as pl
from jax.experimental.pallas import tpu as pltpu
from jax.experimental.pallas import tpu_sc as plsc

K = 16
C = 3
B_, N_ = 8, 4096
RB = 128
CHUNK = 128
NCH = N_ // CHUNK
MAXI = 0x7FFFFFFF


def _topk_body(vall_ref, vrow_ref, idx_ref):
    va = vall_ref[0]
    vb = vrow_ref[0]
    g = lax.dot_general(va, vb, (((1,), (1,)), ((), ())),
                        preferred_element_type=jnp.float32)
    sqa = jnp.sum(va * va, axis=1, keepdims=True)
    sqb = jnp.sum(vb * vb, axis=1)[None, :]
    dist = sqa - 2.0 * g + sqb
    bits = lax.bitcast_convert_type(jnp.maximum(dist, 0.0), jnp.int32)
    b3 = bits.reshape(NCH, CHUNK, RB)
    li3 = lax.broadcasted_iota(jnp.int32, (NCH, CHUNK, RB), 1)
    p = (b3 & jnp.int32(-CHUNK)) | li3
    fi = lax.broadcasted_iota(jnp.int32, (NCH, CHUNK, RB), 0) * CHUNK + li3
    ci = lax.broadcasted_iota(jnp.int32, (NCH, RB), 0)
    ti = lax.broadcasted_iota(jnp.int32, (K, RB), 0)
    self_idx = pl.program_id(1) * RB + lax.broadcasted_iota(
        jnp.int32, (1, RB), 1)
    p = jnp.where(fi == self_idx.reshape(1, 1, RB), jnp.int32(MAXI), p)

    def it(t, carry):
        p, acc = carry
        m2 = jnp.min(p, axis=1)
        m = jnp.min(m2, axis=0, keepdims=True)
        cstar = jnp.min(jnp.where(m2 == m, ci, jnp.int32(MAXI)),
                        axis=0, keepdims=True)
        gidx = cstar * CHUNK + (m & (CHUNK - 1))
        p = jnp.where(fi == gidx.reshape(1, 1, RB), jnp.int32(MAXI), p)
        acc = jnp.where(ti == t, gidx, acc)
        return (p, acc)

    _, acc = lax.fori_loop(0, K, it, (p, jnp.zeros((K, RB), jnp.int32)))
    idx_ref[0] = acc


def _tc_topk(vertices):
    b, n, _ = vertices.shape
    return pl.pallas_call(
        _topk_body,
        grid=(b, n // RB),
        in_specs=[
            pl.BlockSpec((1, n, C), lambda i, j: (i, 0, 0)),
            pl.BlockSpec((1, RB, C), lambda i, j: (i, j, 0)),
        ],
        out_specs=pl.BlockSpec((1, K, RB), lambda i, j: (i, 0, j)),
        out_shape=jax.ShapeDtypeStruct((b, K, n), jnp.int32),
    )(vertices, vertices)


ROWS_PER_W = N_ * B_ // 32
SUB = 256
NSUB = ROWS_PER_W // SUB


def _sc_gather(v_flat, idx_t):
    mesh = plsc.VectorSubcoreMesh(core_axis_name="c", subcore_axis_name="s")

    @functools.partial(
        pl.kernel,
        mesh=mesh,
        out_type=jax.ShapeDtypeStruct((B_ * N_ * K * 2 * C,), jnp.float32),
        compiler_params=pltpu.CompilerParams(needs_layout_passes=False),
        scratch_types=[
            pltpu.VMEM((N_ * C,), jnp.float32),
            pltpu.VMEM((K, SUB), jnp.int32),
            pltpu.VMEM((SUB * K * 2 * C,), jnp.float32),
        ],
    )
    def body(v_hbm, idx_hbm, out_hbm, vf, idxb, outb):
        wid = lax.axis_index("c") * 16 + lax.axis_index("s")
        b = wid // 4
        q = wid % 4
        pltpu.sync_copy(v_hbm.at[pl.ds(b * N_ * C, N_ * C)], vf)
        i6 = lax.iota(jnp.int32, 16) * jnp.int32(2 * C)
        t_iota = lax.iota(jnp.int32, 16)

        for s in range(NSUB):
            row0 = q * ROWS_PER_W + s * SUB
            pltpu.sync_copy(idx_hbm.at[b, pl.ds(0, K), pl.ds(row0, SUB)],
                            idxb)

            def rb(r, carry):
                rv = jnp.broadcast_to(r, (16,)).astype(jnp.int32)
                iv3 = plsc.load_gather(idxb, [t_iota, rv]) * 3
                civ3 = jnp.broadcast_to((row0 + r) * 3, (16,)).astype(
                    jnp.int32)
                nx = plsc.load_gather(vf, [iv3])
                ny = plsc.load_gather(vf, [iv3 + 1])
                nz = plsc.load_gather(vf, [iv3 + 2])
                cx = plsc.load_gather(vf, [civ3])
                cy = plsc.load_gather(vf, [civ3 + 1])
                cz = plsc.load_gather(vf, [civ3 + 2])
                off = r * jnp.int32(K * 2 * C) + i6
                plsc.store_scatter(outb, [off + 0], nx - cx)
                plsc.store_scatter(outb, [off + 1], ny - cy)
                plsc.store_scatter(outb, [off + 2], nz - cz)
                plsc.store_scatter(outb, [off + 3], cx)
                plsc.store_scatter(outb, [off + 4], cy)
                plsc.store_scatter(outb, [off + 5], cz)
                return carry

            lax.fori_loop(0, SUB, rb, 0)
            goff = (b * N_ + row0) * K * 2 * C
            pltpu.sync_copy(outb, out_hbm.at[pl.ds(goff, SUB * K * 2 * C)])

    return body(v_flat, idx_t)


def kernel(vertices):
    b, n, c = vertices.shape
    idx_t = _tc_topk(vertices)
    out_flat = _sc_gather(vertices.reshape(-1), idx_t)
    return out_flat.reshape(b, n, K, 2 * c)
